# baseline (device time: 260786 ns/iter reference)
import jax
import jax.numpy as jnp
from jax import lax
from jax.experimental import pallas as pl
from jax.experimental.pallas import tpu as pltpu

N_DEV = 4
SQ = 2048
SKV = 2048
HQ_TOTAL = 32
HG = 8
DH = 128
D_MODEL = 1024
D_HID = HG * DH
SCALE = 0.08838834764831843
BLK = 64
N_RES = 4
BLKS_PER_RES = (SQ // BLK) // N_RES
RROWS = BLKS_PER_RES * BLK


def kernel(x, Wq, K_ext, V_ext, Wo):
    x2 = x.reshape(SQ, D_MODEL).astype(jnp.bfloat16)
    wq = Wq.astype(jnp.bfloat16)
    wo = Wo.astype(jnp.bfloat16)
    k2 = K_ext.reshape(SKV, HQ_TOTAL * DH).astype(jnp.bfloat16)
    v2 = V_ext.reshape(SKV, HQ_TOTAL * DH).astype(jnp.bfloat16)

    def body(x_ref, wq_ref, k_ref, v_ref, wo_ref, out_ref,
             comm_ref, send_sems, recv_sems):
        my = lax.axis_index("i")
        left = lax.rem(my + N_DEV - 1, N_DEV)
        right = lax.rem(my + 1, N_DEV)

        barrier = pltpu.get_barrier_semaphore()
        for nbr in (left, right):
            pl.semaphore_signal(barrier, inc=1, device_id=(nbr,),
                                device_id_type=pl.DeviceIdType.MESH)
        pl.semaphore_wait(barrier, 2)

        comm_ref[0, :D_MODEL, :] = wq_ref[...]
        comm_ref[0, D_MODEL:, :] = wo_ref[...]

        def compute(h, slot):
            g = lax.rem(my - h + N_DEV, N_DEV)
            col0 = g * D_HID
            wq_g = comm_ref[slot, :D_MODEL, :]
            wo_g = comm_ref[slot, D_MODEL:, :]
            for r in range(N_RES):
                rows = [(r + N_RES * m) * BLK for m in range(BLKS_PER_RES)]
                xr = jnp.concatenate(
                    [x_ref[pl.ds(o, BLK), :] for o in rows], axis=0)
                qr = jnp.dot(xr, wq_g,
                             preferred_element_type=jnp.float32
                             ).astype(jnp.bfloat16)
                kr = jnp.concatenate(
                    [k_ref[pl.ds(o, BLK), pl.ds(col0, D_HID)] for o in rows],
                    axis=0)
                vr = jnp.concatenate(
                    [v_ref[pl.ds(o, BLK), pl.ds(col0, D_HID)] for o in rows],
                    axis=0)
                ctx_cols = []
                for hh in range(HG):
                    c = hh * DH
                    qh = qr[:, c:c + DH]
                    kh = kr[:, c:c + DH]
                    s = lax.dot_general(
                        qh, kh, (((1,), (1,)), ((), ())),
                        preferred_element_type=jnp.float32) * SCALE
                    s = s - jnp.max(s, axis=1, keepdims=True)
                    w = jnp.exp(s)
                    w = (w / jnp.sum(w, axis=1, keepdims=True)
                         ).astype(jnp.bfloat16)
                    ctx_cols.append(
                        jnp.dot(w, vr[:, c:c + DH],
                                preferred_element_type=jnp.float32
                                ).astype(jnp.bfloat16))
                ctx_r = jnp.concatenate(ctx_cols, axis=1)
                part = jnp.dot(ctx_r, wo_g,
                               preferred_element_type=jnp.float32)
                for m in range(BLKS_PER_RES):
                    o = (r + N_RES * m) * BLK
                    blk = part[m * BLK:(m + 1) * BLK, :]
                    if h == 0:
                        out_ref[pl.ds(o, BLK), :] = blk
                    else:
                        out_ref[pl.ds(o, BLK), :] += blk

        for h in range(N_DEV):
            slot = h % 2
            if h < N_DEV - 1:
                rdma = pltpu.make_async_remote_copy(
                    src_ref=comm_ref.at[slot],
                    dst_ref=comm_ref.at[1 - slot],
                    send_sem=send_sems.at[slot],
                    recv_sem=recv_sems.at[1 - slot],
                    device_id=(right,),
                    device_id_type=pl.DeviceIdType.MESH,
                )
                rdma.start()
                compute(h, slot)
                rdma.wait()
            else:
                compute(h, slot)

    out = pl.pallas_call(
        body,
        out_shape=jax.ShapeDtypeStruct((SQ, D_MODEL), jnp.float32),
        in_specs=[pl.BlockSpec(memory_space=pltpu.VMEM)] * 5,
        out_specs=pl.BlockSpec(memory_space=pltpu.VMEM),
        scratch_shapes=[
            pltpu.VMEM((2, 2 * D_MODEL, D_MODEL), jnp.bfloat16),
            pltpu.SemaphoreType.DMA((2,)),
            pltpu.SemaphoreType.DMA((2,)),
        ],
        compiler_params=pltpu.CompilerParams(
            collective_id=0,
            vmem_limit_bytes=128 * 1024 * 1024,
        ),
    )(x2, wq, k2, v2, wo)
    return out.reshape(1, SQ, D_MODEL)
